# Initial kernel scaffold; baseline (speedup 1.0000x reference)
import jax
import jax.numpy as jnp
from jax.experimental import pallas as pl


def kernel(bond_attr, emb0, emb1, emb2):
    raise NotImplementedError("write your pallas kernel here")


# speedup vs baseline: 4.9560x; 4.9560x over previous
import functools

import jax
import jax.numpy as jnp
from jax import lax
from jax.experimental import pallas as pl
from jax.experimental.pallas import tpu as pltpu
from jax.experimental.pallas import tpu_sc as plsc

_NB = 320000
_NT = 6
_H = 128
_NCOMB = _NT * _NT * _NT
_NW = 32
_NB_W = _NB // _NW
_CHUNK = 80
_NCHUNK = _NB_W // _CHUNK


def _build_combined(e0_ref, e1_ref, e2_ref, c_ref):
    e2 = e2_ref[...]
    for i0 in range(_NT):
        r0 = e0_ref[i0, :][None, :]
        for i1 in range(_NT):
            r1 = e1_ref[i1, :][None, :]
            c_ref[pl.ds((i0 * _NT + i1) * _NT, _NT), :] = e2 + r0 + r1


def _fuse_idx(a0_ref, a1_ref, a2_ref, o_ref):
    o_ref[...] = a0_ref[...] * (_NT * _NT) + a1_ref[...] * _NT + a2_ref[...]


@functools.partial(
    pl.kernel,
    out_type=jax.ShapeDtypeStruct((_NB, _H), jnp.float32),
    mesh=plsc.VectorSubcoreMesh(core_axis_name="c", subcore_axis_name="s"),
    scratch_types=[
        pltpu.VMEM((_NB_W,), jnp.int32),
        pltpu.VMEM((_CHUNK, _H), jnp.float32),
        pltpu.SemaphoreType.DMA,
    ],
)
def _sc_gather(c_hbm, idx_hbm, out_hbm, idx_v, buf_v, sem):
    wid = lax.axis_index("s") * 2 + lax.axis_index("c")
    base = wid * _NB_W
    pltpu.sync_copy(idx_hbm.at[pl.ds(base, _NB_W)], idx_v)

    def chunk_body(c, carry):
        pltpu.async_copy(
            c_hbm.at[idx_v.at[pl.ds(c * _CHUNK, _CHUNK)]], buf_v, sem
        ).wait()
        pltpu.sync_copy(buf_v, out_hbm.at[pl.ds(base + c * _CHUNK, _CHUNK)])
        return carry

    lax.fori_loop(0, _NCHUNK, chunk_body, 0)


def kernel(bond_attr, emb0, emb1, emb2):
    a = bond_attr.astype(jnp.int32)
    a0 = a[:, 0].reshape(_NB // _H, _H)
    a1 = a[:, 1].reshape(_NB // _H, _H)
    a2 = a[:, 2].reshape(_NB // _H, _H)

    comb = pl.pallas_call(
        _build_combined,
        out_shape=jax.ShapeDtypeStruct((_NCOMB, _H), jnp.float32),
    )(emb0, emb1, emb2)

    idx = pl.pallas_call(
        _fuse_idx,
        out_shape=jax.ShapeDtypeStruct((_NB // _H, _H), jnp.int32),
    )(a0, a1, a2)

    return _sc_gather(comb, idx.reshape(_NB))


# speedup vs baseline: 5.0206x; 1.0130x over previous
import functools

import jax
import jax.numpy as jnp
from jax import lax
from jax.experimental import pallas as pl
from jax.experimental.pallas import tpu as pltpu
from jax.experimental.pallas import tpu_sc as plsc

_NB = 320000
_NT = 6
_H = 128
_NCOMB = _NT * _NT * _NT
_NW = 32
_CH = 128
_NCH = 78
_NEXTRA = (_NB - _NW * _NCH * _CH) // _CH


def _build_combined(e0_ref, e1_ref, e2_ref, c_ref):
    e2 = e2_ref[...]
    for i0 in range(_NT):
        r0 = e0_ref[i0, :][None, :]
        for i1 in range(_NT):
            r1 = e1_ref[i1, :][None, :]
            c_ref[pl.ds((i0 * _NT + i1) * _NT, _NT), :] = e2 + r0 + r1


def _fuse_idx(a0_ref, a1_ref, a2_ref, o_ref):
    o_ref[...] = a0_ref[...] * (_NT * _NT) + a1_ref[...] * _NT + a2_ref[...]


@functools.partial(
    pl.kernel,
    out_type=jax.ShapeDtypeStruct((_NB, _H), jnp.float32),
    mesh=plsc.VectorSubcoreMesh(core_axis_name="c", subcore_axis_name="s"),
    scratch_types=[
        pltpu.VMEM((_NCH * _CH,), jnp.int32),
        pltpu.VMEM((_CH,), jnp.int32),
        pltpu.VMEM((_CH, _H), jnp.float32),
        pltpu.VMEM((_CH, _H), jnp.float32),
        pltpu.SemaphoreType.DMA,
        pltpu.SemaphoreType.DMA,
        pltpu.SemaphoreType.DMA,
        pltpu.SemaphoreType.DMA,
    ],
)
def _sc_gather(c_hbm, idx_hbm, out_hbm, idx_v, idx_x, buf0, buf1,
               sg0, sg1, ss0, ss1):
    wid = lax.axis_index("s") * 2 + lax.axis_index("c")
    row0 = wid * (_NCH * _CH)
    bufs = (buf0, buf1)
    sgs = (sg0, sg1)
    sss = (ss0, ss1)

    pltpu.sync_copy(idx_hbm.at[pl.ds(row0, _NCH * _CH)], idx_v)

    def gstart(c, b):
        pltpu.async_copy(
            c_hbm.at[idx_v.at[pl.ds(c * _CH, _CH)]], bufs[b], sgs[b])

    def gwait(c, b):
        pltpu.make_async_copy(
            c_hbm.at[idx_v.at[pl.ds(c * _CH, _CH)]], bufs[b], sgs[b]).wait()

    def sstart(c, b):
        pltpu.async_copy(
            bufs[b], out_hbm.at[pl.ds(row0 + c * _CH, _CH)], sss[b])

    def swait(c, b):
        pltpu.make_async_copy(
            bufs[b], out_hbm.at[pl.ds(row0 + c * _CH, _CH)], sss[b]).wait()

    gstart(0, 0)

    def pair_body(g, carry):
        for b in range(2):
            c = g * 2 + b
            @pl.when(c > 0)
            def _():
                swait(c - 1, 1 - b)

            @pl.when(c + 1 < _NCH)
            def _():
                gstart(c + 1, 1 - b)

            gwait(c, b)
            sstart(c, b)
        return carry

    lax.fori_loop(0, _NCH // 2, pair_body, 0)
    swait(_NCH - 1, 1)

    @pl.when(wid < _NEXTRA)
    def _():
        xrow = _NW * _NCH * _CH + wid * _CH
        pltpu.sync_copy(idx_hbm.at[pl.ds(xrow, _CH)], idx_x)
        pltpu.async_copy(c_hbm.at[idx_x], buf0, sg0).wait()
        pltpu.sync_copy(buf0, out_hbm.at[pl.ds(xrow, _CH)])


def kernel(bond_attr, emb0, emb1, emb2):
    a = bond_attr.astype(jnp.int32)
    a0 = a[:, 0].reshape(_NB // _H, _H)
    a1 = a[:, 1].reshape(_NB // _H, _H)
    a2 = a[:, 2].reshape(_NB // _H, _H)

    comb = pl.pallas_call(
        _build_combined,
        out_shape=jax.ShapeDtypeStruct((_NCOMB, _H), jnp.float32),
    )(emb0, emb1, emb2)

    idx = pl.pallas_call(
        _fuse_idx,
        out_shape=jax.ShapeDtypeStruct((_NB // _H, _H), jnp.int32),
    )(a0, a1, a2)

    return _sc_gather(comb, idx.reshape(_NB))


# speedup vs baseline: 18.9300x; 3.7705x over previous
import functools

import jax
import jax.numpy as jnp
from jax import lax
from jax.experimental import pallas as pl
from jax.experimental.pallas import tpu as pltpu
from jax.experimental.pallas import tpu_sc as plsc

_NB = 320000
_NT = 6
_H = 128
_NCOMB = _NT * _NT * _NT
_NW = 32
_CH = 128
_NCH = 78
_NEXTRA = (_NB - _NW * _NCH * _CH) // _CH


def _build_combined(e0_ref, e1_ref, e2_ref, c_ref):
    e2 = e2_ref[...]
    for i0 in range(_NT):
        r0 = e0_ref[i0, :][None, :]
        for i1 in range(_NT):
            r1 = e1_ref[i1, :][None, :]
            c_ref[pl.ds((i0 * _NT + i1) * _NT, _NT), :] = e2 + r0 + r1


def _fuse_idx(a0_ref, a1_ref, a2_ref, o_ref):
    o_ref[...] = a0_ref[...] * (_NT * _NT) + a1_ref[...] * _NT + a2_ref[...]


@functools.partial(
    pl.kernel,
    out_type=jax.ShapeDtypeStruct((_NB, _H), jnp.float32),
    mesh=plsc.VectorSubcoreMesh(core_axis_name="c", subcore_axis_name="s"),
    scratch_types=[
        pltpu.VMEM_SHARED((_NCOMB, _H), jnp.float32),
        pltpu.VMEM((_NCH * _CH,), jnp.int32),
        pltpu.VMEM((_CH,), jnp.int32),
        pltpu.VMEM((_CH, _H), jnp.float32),
        pltpu.VMEM((_CH, _H), jnp.float32),
        pltpu.VMEM((_CH, _H), jnp.float32),
        pltpu.VMEM((_CH, _H), jnp.float32),
        pltpu.SemaphoreType.DMA,
        pltpu.SemaphoreType.DMA,
        pltpu.SemaphoreType.DMA,
        pltpu.SemaphoreType.DMA,
        pltpu.SemaphoreType.DMA,
        pltpu.SemaphoreType.DMA,
        pltpu.SemaphoreType.DMA,
        pltpu.SemaphoreType.DMA,
    ],
)
def _sc_gather(c_hbm, idx_hbm, out_hbm, c_sp, idx_v, idx_x,
               buf0, buf1, buf2, buf3,
               sg0, sg1, sg2, sg3, ss0, ss1, ss2, ss3):
    sub = lax.axis_index("s")
    wid = sub * 2 + lax.axis_index("c")
    row0 = wid * (_NCH * _CH)
    bufs = (buf0, buf1, buf2, buf3)
    sgs = (sg0, sg1, sg2, sg3)
    sss = (ss0, ss1, ss2, ss3)

    @pl.when(sub == 0)
    def _():
        pltpu.sync_copy(c_hbm, c_sp)

    pltpu.sync_copy(idx_hbm.at[pl.ds(row0, _NCH * _CH)], idx_v)

    plsc.subcore_barrier()

    def gstart(c, b):
        pltpu.async_copy(
            c_sp.at[idx_v.at[pl.ds(c * _CH, _CH)]], bufs[b], sgs[b])

    def gwait(c, b):
        pltpu.make_async_copy(
            c_sp.at[idx_v.at[pl.ds(c * _CH, _CH)]], bufs[b], sgs[b]).wait()

    def sstart(c, b):
        pltpu.async_copy(
            bufs[b], out_hbm.at[pl.ds(row0 + c * _CH, _CH)], sss[b])

    def swait(c, b):
        pltpu.make_async_copy(
            bufs[b], out_hbm.at[pl.ds(row0 + c * _CH, _CH)], sss[b]).wait()

    gstart(0, 0)
    gstart(1, 1)

    def quad_body(g, carry):
        for d in range(4):
            c = g * 4 + d
            @pl.when(c >= 2)
            def _():
                swait(c - 2, (d + 2) % 4)

            gstart(c + 2, (d + 2) % 4)
            gwait(c, d)
            sstart(c, d)
        return carry

    lax.fori_loop(0, (_NCH - 2) // 4, quad_body, 0)

    swait(74, 2)
    gwait(76, 0)
    sstart(76, 0)
    swait(75, 3)
    gwait(77, 1)
    sstart(77, 1)
    swait(76, 0)
    swait(77, 1)

    @pl.when(wid < _NEXTRA)
    def _():
        xrow = _NW * _NCH * _CH + wid * _CH
        pltpu.sync_copy(idx_hbm.at[pl.ds(xrow, _CH)], idx_x)
        pltpu.async_copy(c_sp.at[idx_x], buf0, sg0).wait()
        pltpu.sync_copy(buf0, out_hbm.at[pl.ds(xrow, _CH)])


def kernel(bond_attr, emb0, emb1, emb2):
    a = bond_attr.astype(jnp.int32)
    a0 = a[:, 0].reshape(_NB // _H, _H)
    a1 = a[:, 1].reshape(_NB // _H, _H)
    a2 = a[:, 2].reshape(_NB // _H, _H)

    comb = pl.pallas_call(
        _build_combined,
        out_shape=jax.ShapeDtypeStruct((_NCOMB, _H), jnp.float32),
    )(emb0, emb1, emb2)

    idx = pl.pallas_call(
        _fuse_idx,
        out_shape=jax.ShapeDtypeStruct((_NB // _H, _H), jnp.int32),
    )(a0, a1, a2)

    return _sc_gather(comb, idx.reshape(_NB))
